# baseline (speedup 1.0000x reference)
import functools

import jax
import jax.numpy as jnp
from jax.experimental import pallas as pl
from jax.experimental.pallas import tpu as pltpu


def _gcn_body(a_ref, feat_ref, w_ref, adj_ref, bias_ref, out_ref, h_ref, *, bk):
    i = pl.program_id(0)
    k = pl.program_id(1)
    nk = pl.num_programs(1)

    @pl.when((i == 0) & (k == 0))
    def _():
        h_ref[...] = jax.lax.dot_general(
            feat_ref[...], w_ref[...],
            dimension_numbers=(((1,), (1,)), ((), ())),
            preferred_element_type=jnp.float32,
        )

    part = jax.lax.dot_general(
        adj_ref[...], h_ref[pl.ds(k * bk, bk), :],
        dimension_numbers=(((1,), (0,)), ((), ())),
        preferred_element_type=jnp.float32,
    )

    @pl.when(k == 0)
    def _():
        out_ref[...] = part

    @pl.when(k > 0)
    def _():
        out_ref[...] += part

    @pl.when(k == nk - 1)
    def _():
        out = out_ref[...] + bias_ref[...]
        alpha = a_ref[0, 0]
        out_ref[...] = jnp.where(out >= 0, out, alpha * out)


@functools.partial(jax.jit, static_argnames=("bm", "bk"))
def _gcn(feat2, adj2, W, bias2, a2, bm, bk):
    n, d_in = feat2.shape
    d_out = W.shape[0]

    return pl.pallas_call(
        functools.partial(_gcn_body, bk=bk),
        grid=(n // bm, n // bk),
        in_specs=[
            pl.BlockSpec(memory_space=pltpu.SMEM),
            pl.BlockSpec((n, d_in), lambda i, k: (0, 0)),
            pl.BlockSpec((d_out, d_in), lambda i, k: (0, 0)),
            pl.BlockSpec((bm, bk), lambda i, k: (i, k)),
            pl.BlockSpec((1, d_out), lambda i, k: (0, 0)),
        ],
        out_specs=pl.BlockSpec((bm, d_out), lambda i, k: (i, 0)),
        out_shape=jax.ShapeDtypeStruct((n, d_out), jnp.float32),
        scratch_shapes=[pltpu.VMEM((n, d_out), jnp.float32)],
        compiler_params=pltpu.CompilerParams(
            dimension_semantics=("arbitrary", "arbitrary"),
        ),
    )(a2, feat2, W, adj2, bias2)


def kernel(feat, adj, W, bias, prelu_a):
    b, n, d_in = feat.shape
    d_out = W.shape[0]
    feat2 = feat.reshape(n, d_in)
    adj2 = adj.reshape(n, n)
    bias2 = bias.reshape(1, d_out)
    a2 = jnp.asarray(prelu_a, jnp.float32).reshape(1, 1)
    if n % 512 == 0 and n % 4096 == 0:
        bm, bk = 512, 4096
    else:
        bm, bk = n, n
    out = _gcn(feat2, adj2, W, bias2, a2, bm, bk)
    return out.reshape(b, n, d_out)


# speedup vs baseline: 1.0055x; 1.0055x over previous
import functools

import jax
import jax.numpy as jnp
from jax.experimental import pallas as pl
from jax.experimental.pallas import tpu as pltpu


def _gcn_body(a_ref, feat_ref, w_ref, adj_ref, bias_ref, out_ref, h_ref):
    @pl.when(pl.program_id(0) == 0)
    def _():
        h_ref[...] = jax.lax.dot_general(
            feat_ref[...], w_ref[...],
            dimension_numbers=(((1,), (1,)), ((), ())),
            preferred_element_type=jnp.float32,
        )

    out = jax.lax.dot_general(
        adj_ref[...], h_ref[...],
        dimension_numbers=(((1,), (0,)), ((), ())),
        preferred_element_type=jnp.float32,
    )
    out = out + bias_ref[...]
    alpha = a_ref[0, 0]
    out_ref[...] = jnp.where(out >= 0, out, alpha * out)


@functools.partial(jax.jit, static_argnames=("bm",))
def _gcn(feat2, adj2, W, bias2, a2, bm):
    n, d_in = feat2.shape
    d_out = W.shape[0]

    return pl.pallas_call(
        _gcn_body,
        grid=(n // bm,),
        in_specs=[
            pl.BlockSpec(memory_space=pltpu.SMEM),
            pl.BlockSpec((n, d_in), lambda i: (0, 0)),
            pl.BlockSpec((d_out, d_in), lambda i: (0, 0)),
            pl.BlockSpec((bm, n), lambda i: (i, 0)),
            pl.BlockSpec((1, d_out), lambda i: (0, 0)),
        ],
        out_specs=pl.BlockSpec((bm, d_out), lambda i: (i, 0)),
        out_shape=jax.ShapeDtypeStruct((n, d_out), jnp.float32),
        scratch_shapes=[pltpu.VMEM((n, d_out), jnp.float32)],
        compiler_params=pltpu.CompilerParams(
            dimension_semantics=("arbitrary",),
        ),
    )(a2, feat2, W, adj2, bias2)


def kernel(feat, adj, W, bias, prelu_a):
    b, n, d_in = feat.shape
    d_out = W.shape[0]
    feat2 = feat.reshape(n, d_in)
    adj2 = adj.reshape(n, n)
    bias2 = bias.reshape(1, d_out)
    a2 = jnp.asarray(prelu_a, jnp.float32).reshape(1, 1)
    bm = 256 if n % 256 == 0 else n
    out = _gcn(feat2, adj2, W, bias2, a2, bm)
    return out.reshape(b, n, d_out)


# speedup vs baseline: 1.0151x; 1.0096x over previous
import functools

import jax
import jax.numpy as jnp
from jax.experimental import pallas as pl
from jax.experimental.pallas import tpu as pltpu


def _gcn_body(a_ref, feat_ref, w_ref, adj_ref, bias_ref, out_ref, h_ref, *, bk):
    i = pl.program_id(0)
    k = pl.program_id(1)
    nk = pl.num_programs(1)

    @pl.when((i == 0) & (k == 0))
    def _():
        h_ref[...] = jax.lax.dot_general(
            feat_ref[...], w_ref[...],
            dimension_numbers=(((1,), (1,)), ((), ())),
            preferred_element_type=jnp.float32,
        )

    part = jax.lax.dot_general(
        adj_ref[...], h_ref[pl.ds(k * bk, bk), :],
        dimension_numbers=(((1,), (0,)), ((), ())),
        preferred_element_type=jnp.float32,
    )

    @pl.when(k == 0)
    def _():
        out_ref[...] = part

    @pl.when(k > 0)
    def _():
        out_ref[...] += part

    @pl.when(k == nk - 1)
    def _():
        out = out_ref[...] + bias_ref[...]
        alpha = a_ref[0, 0]
        out_ref[...] = jnp.where(out >= 0, out, alpha * out)


@functools.partial(jax.jit, static_argnames=("bm", "bk"))
def _gcn(feat2, adj2, W, bias2, a2, bm, bk):
    n, d_in = feat2.shape
    d_out = W.shape[0]

    return pl.pallas_call(
        functools.partial(_gcn_body, bk=bk),
        grid=(n // bm, n // bk),
        in_specs=[
            pl.BlockSpec(memory_space=pltpu.SMEM),
            pl.BlockSpec((n, d_in), lambda i, k: (0, 0)),
            pl.BlockSpec((d_out, d_in), lambda i, k: (0, 0)),
            pl.BlockSpec((bm, bk), lambda i, k: (i, k)),
            pl.BlockSpec((1, d_out), lambda i, k: (0, 0)),
        ],
        out_specs=pl.BlockSpec((bm, d_out), lambda i, k: (i, 0)),
        out_shape=jax.ShapeDtypeStruct((n, d_out), jnp.float32),
        scratch_shapes=[pltpu.VMEM((n, d_out), jnp.float32)],
        compiler_params=pltpu.CompilerParams(
            dimension_semantics=("arbitrary", "arbitrary"),
        ),
    )(a2, feat2, W, adj2, bias2)


def kernel(feat, adj, W, bias, prelu_a):
    b, n, d_in = feat.shape
    d_out = W.shape[0]
    feat2 = feat.reshape(n, d_in)
    adj2 = adj.reshape(n, n)
    bias2 = bias.reshape(1, d_out)
    a2 = jnp.asarray(prelu_a, jnp.float32).reshape(1, 1)
    if n % 1024 == 0 and n % 4096 == 0:
        bm, bk = 1024, 4096
    else:
        bm, bk = n, n
    out = _gcn(feat2, adj2, W, bias2, a2, bm, bk)
    return out.reshape(b, n, d_out)
